# baseline (speedup 1.0000x reference)
import functools

import jax
import jax.numpy as jnp
from jax.experimental import pallas as pl
from jax.experimental.pallas import tpu as pltpu

_LN_EPS = 1e-5


def _fused_kernel(p_hbm, f_hbm, a_hbm, w_in_ref, b_in_ref,
                  g_in_ref, be_in_ref, g_out_ref, be_out_ref,
                  w_head_ref, b_head_ref, out_ref,
                  pbuf, fbuf, abuf, z_ref, psem, fsem, asem,
                  *, NC, G, H, N, d_in, d_model):
    GH = G * H
    GHN = G * H * N

    def fa_copies(k):
        return (
            pltpu.make_async_copy(f_hbm.at[pl.ds(k * GHN, GHN)],
                                  fbuf.at[k], fsem.at[k]),
            pltpu.make_async_copy(a_hbm.at[pl.ds(k * GHN, GHN)],
                                  abuf.at[k], asem.at[k]),
        )

    def p_copy(k):
        return pltpu.make_async_copy(p_hbm.at[pl.ds(k * GH, GH)],
                                     pbuf.at[k], psem.at[k])

    for k in range(NC):
        for c in fa_copies(k):
            c.start()
        p_copy(k).start()

    for k in range(NC):
        for c in fa_copies(k):
            c.wait()

        f = fbuf[k]
        a = abuf[k]
        y = (jnp.dot(f, w_in_ref[0:d_in, :],
                     preferred_element_type=jnp.float32)
             + jnp.dot(a, w_in_ref[d_in:2 * d_in, :],
                       preferred_element_type=jnp.float32)
             + b_in_ref[...])

        mu = jnp.mean(y, axis=-1, keepdims=True)
        var = jnp.mean(y * y, axis=-1, keepdims=True) - mu * mu
        y = ((y - mu) * jax.lax.rsqrt(var + _LN_EPS) * g_in_ref[...]
             + be_in_ref[...])
        y = jnp.maximum(y, 0.0)

        p_copy(k).wait()

        for g in range(G):
            for h in range(H):
                i = g * H + h
                p = pbuf[k, i]
                sf = jnp.dot(p, y[i * N:(i + 1) * N, :],
                             preferred_element_type=jnp.float32)
                ob = jax.lax.dot_general(p, sf, (((0,), (0,)), ((), ())),
                                         preferred_element_type=jnp.float32)
                z_ref[g * N:(g + 1) * N,
                      h * d_model:(h + 1) * d_model] = ob

        z = z_ref[...]
        mu = jnp.mean(z, axis=-1, keepdims=True)
        var = jnp.mean(z * z, axis=-1, keepdims=True) - mu * mu
        zn = (z - mu) * jax.lax.rsqrt(var + _LN_EPS) * g_out_ref[...] + be_out_ref[...]
        out_ref[k * G * N:(k + 1) * G * N, :] = (
            jnp.dot(zn, w_head_ref[...], preferred_element_type=jnp.float32)
            + b_head_ref[...])


def kernel(perm, adj, features, appd, w_in, b_in, ln_in_g, ln_in_b,
           ln_out_g, ln_out_b, w_head, b_head):
    del adj
    B, H, N, _ = perm.shape
    d_in = features.shape[-1]
    d_model = w_in.shape[1]
    nclass = w_head.shape[1]

    G = min(8, B)
    NC = B // G

    p2 = perm.reshape(B * H, N, N)
    f2 = features.reshape(B * H * N, d_in)
    a2 = appd.reshape(B * H * N, d_in)

    fused = functools.partial(_fused_kernel, NC=NC, G=G, H=H, N=N,
                              d_in=d_in, d_model=d_model)
    out = pl.pallas_call(
        fused,
        out_shape=jax.ShapeDtypeStruct((B * N, nclass), jnp.float32),
        grid=(1,),
        in_specs=[
            pl.BlockSpec(memory_space=pl.ANY),
            pl.BlockSpec(memory_space=pl.ANY),
            pl.BlockSpec(memory_space=pl.ANY),
            pl.BlockSpec((2 * d_in, d_model), lambda c: (0, 0)),
            pl.BlockSpec((1, d_model), lambda c: (0, 0)),
            pl.BlockSpec((1, d_model), lambda c: (0, 0)),
            pl.BlockSpec((1, d_model), lambda c: (0, 0)),
            pl.BlockSpec((1, H * d_model), lambda c: (0, 0)),
            pl.BlockSpec((1, H * d_model), lambda c: (0, 0)),
            pl.BlockSpec((H * d_model, nclass), lambda c: (0, 0)),
            pl.BlockSpec((1, nclass), lambda c: (0, 0)),
        ],
        out_specs=pl.BlockSpec((B * N, nclass), lambda c: (0, 0)),
        scratch_shapes=[
            pltpu.VMEM((B // G, G * H, N, N), jnp.float32),
            pltpu.VMEM((B // G, G * H * N, d_in), jnp.float32),
            pltpu.VMEM((B // G, G * H * N, d_in), jnp.float32),
            pltpu.VMEM((G * N, H * d_model), jnp.float32),
            pltpu.SemaphoreType.DMA((B // G,)),
            pltpu.SemaphoreType.DMA((B // G,)),
            pltpu.SemaphoreType.DMA((B // G,)),
        ],
        compiler_params=pltpu.CompilerParams(
            dimension_semantics=("arbitrary",)),
    )(p2, f2, a2, w_in, b_in, ln_in_g, ln_in_b,
      ln_out_g, ln_out_b, w_head, b_head)
    return out.reshape(B, N, nclass)


# speedup vs baseline: 1.1143x; 1.1143x over previous
import functools

import jax
import jax.numpy as jnp
from jax.experimental import pallas as pl
from jax.experimental.pallas import tpu as pltpu

_LN_EPS = 1e-5


def _fused_kernel(p_hbm, f_hbm, a_hbm, w_in_ref, b_in_ref,
                  g_in_ref, be_in_ref, g_out_ref, be_out_ref,
                  w_head_ref, b_head_ref, out_ref,
                  pbuf, fbuf, abuf, z_ref, obuf, psem, fsem, asem, osem,
                  *, NC, G, H, N, d_in, d_model):
    GH = G * H
    GHN = G * H * N

    def copies(k):
        return (
            pltpu.make_async_copy(p_hbm.at[pl.ds(k * GH, GH)],
                                  pbuf.at[k], psem.at[k]),
            pltpu.make_async_copy(f_hbm.at[pl.ds(k * GHN, GHN)],
                                  fbuf.at[k], fsem.at[k]),
            pltpu.make_async_copy(a_hbm.at[pl.ds(k * GHN, GHN)],
                                  abuf.at[k], asem.at[k]),
        )

    for k in range(NC):
        for c in copies(k):
            c.start()

    for k in range(NC):
        for c in copies(k):
            c.wait()

        f = fbuf[k]
        a = abuf[k]
        y = (jnp.dot(f, w_in_ref[0:d_in, :],
                     preferred_element_type=jnp.float32)
             + jnp.dot(a, w_in_ref[d_in:2 * d_in, :],
                       preferred_element_type=jnp.float32)
             + b_in_ref[...])

        mu = jnp.mean(y, axis=-1, keepdims=True)
        var = jnp.mean(y * y, axis=-1, keepdims=True) - mu * mu
        y = ((y - mu) * jax.lax.rsqrt(var + _LN_EPS) * g_in_ref[...]
             + be_in_ref[...])
        y = jnp.maximum(y, 0.0)

        for g in range(G):
            for h in range(H):
                i = g * H + h
                p = pbuf[k, i]
                sf = jnp.dot(p, y[i * N:(i + 1) * N, :],
                             preferred_element_type=jnp.float32)
                ob = jax.lax.dot_general(p, sf, (((0,), (0,)), ((), ())),
                                         preferred_element_type=jnp.float32)
                z_ref[g * N:(g + 1) * N,
                      h * d_model:(h + 1) * d_model] = ob

        z = z_ref[...]
        mu = jnp.mean(z, axis=-1, keepdims=True)
        var = jnp.mean(z * z, axis=-1, keepdims=True) - mu * mu
        zn = (z - mu) * jax.lax.rsqrt(var + _LN_EPS) * g_out_ref[...] + be_out_ref[...]
        obuf[k] = (
            jnp.dot(zn, w_head_ref[...], preferred_element_type=jnp.float32)
            + b_head_ref[...])
        pltpu.make_async_copy(obuf.at[k],
                              out_ref.at[pl.ds(k * G * N, G * N)],
                              osem.at[k]).start()

    for k in range(NC):
        pltpu.make_async_copy(obuf.at[k],
                              out_ref.at[pl.ds(k * G * N, G * N)],
                              osem.at[k]).wait()


def kernel(perm, adj, features, appd, w_in, b_in, ln_in_g, ln_in_b,
           ln_out_g, ln_out_b, w_head, b_head):
    del adj
    B, H, N, _ = perm.shape
    d_in = features.shape[-1]
    d_model = w_in.shape[1]
    nclass = w_head.shape[1]

    G = min(8, B)
    NC = B // G

    p2 = perm.reshape(B * H, N, N)
    f2 = features.reshape(B * H * N, d_in)
    a2 = appd.reshape(B * H * N, d_in)

    fused = functools.partial(_fused_kernel, NC=NC, G=G, H=H, N=N,
                              d_in=d_in, d_model=d_model)
    out = pl.pallas_call(
        fused,
        out_shape=jax.ShapeDtypeStruct((B * N, nclass), jnp.float32),
        grid=(1,),
        in_specs=[
            pl.BlockSpec(memory_space=pl.ANY),
            pl.BlockSpec(memory_space=pl.ANY),
            pl.BlockSpec(memory_space=pl.ANY),
            pl.BlockSpec((2 * d_in, d_model), lambda c: (0, 0)),
            pl.BlockSpec((1, d_model), lambda c: (0, 0)),
            pl.BlockSpec((1, d_model), lambda c: (0, 0)),
            pl.BlockSpec((1, d_model), lambda c: (0, 0)),
            pl.BlockSpec((1, H * d_model), lambda c: (0, 0)),
            pl.BlockSpec((1, H * d_model), lambda c: (0, 0)),
            pl.BlockSpec((H * d_model, nclass), lambda c: (0, 0)),
            pl.BlockSpec((1, nclass), lambda c: (0, 0)),
        ],
        out_specs=pl.BlockSpec(memory_space=pl.ANY),
        scratch_shapes=[
            pltpu.VMEM((B // G, G * H, N, N), jnp.float32),
            pltpu.VMEM((B // G, G * H * N, d_in), jnp.float32),
            pltpu.VMEM((B // G, G * H * N, d_in), jnp.float32),
            pltpu.VMEM((G * N, H * d_model), jnp.float32),
            pltpu.VMEM((B // G, G * N, nclass), jnp.float32),
            pltpu.SemaphoreType.DMA((B // G,)),
            pltpu.SemaphoreType.DMA((B // G,)),
            pltpu.SemaphoreType.DMA((B // G,)),
            pltpu.SemaphoreType.DMA((B // G,)),
        ],
        compiler_params=pltpu.CompilerParams(
            dimension_semantics=("arbitrary",)),
    )(p2, f2, a2, w_in, b_in, ln_in_g, ln_in_b,
      ln_out_g, ln_out_b, w_head, b_head)
    return out.reshape(B, N, nclass)
